# baseline (speedup 1.0000x reference)
import functools

import jax
import jax.numpy as jnp
from jax import lax
from jax.experimental import pallas as pl
from jax.experimental.pallas import tpu as pltpu
from jax.experimental.pallas import tpu_sc as plsc

N_NODES = 50000
SUBTOK = 20
EMB = 128

NC = 2
NS = 16
UNIT = 392
N_UNITS = 128
N_PAD = N_UNITS * UNIT
U_CORE0 = 0
U_CORE1 = 8
IDX_UNIT = UNIT * SUBTOK

_mesh = plsc.VectorSubcoreMesh(core_axis_name="c", subcore_axis_name="s")


@functools.partial(
    pl.kernel,
    out_type=jax.ShapeDtypeStruct((N_PAD, EMB), jnp.float32),
    mesh=_mesh,
    scratch_types=[
        pltpu.VMEM((IDX_UNIT,), jnp.int32),
        pltpu.VMEM((UNIT,), jnp.int32),
        pltpu.VMEM((UNIT, EMB), jnp.float32),
        pltpu.SemaphoreType.DMA,
        pltpu.SemaphoreType.DMA,
    ],
)
def _node_embedding_sc(tokens_hbm, nodes_hbm, token_table, node_table,
                       out_hbm, tok_idx_v, node_idx_v, acc_v,
                       sem_add, sem_init):
    cid = lax.axis_index("c")
    sid = lax.axis_index("s")
    n_units = lax.select(cid == 0, U_CORE0, U_CORE1)
    unit0 = lax.select(cid == 0, sid * U_CORE0,
                       NS * U_CORE0 + sid * U_CORE1)

    def unit_body(u, _):
        base = (unit0 + u) * UNIT
        pltpu.sync_copy(tokens_hbm.at[pl.ds(base * SUBTOK, IDX_UNIT)],
                        tok_idx_v)
        pltpu.sync_copy(nodes_hbm.at[pl.ds(base, UNIT)], node_idx_v)
        pltpu.async_copy(node_table.at[node_idx_v], acc_v, sem_init).wait()

        def sub_body(j, _):
            pltpu.async_copy(
                token_table.at[tok_idx_v.at[pl.ds(j * UNIT, UNIT)]],
                acc_v, sem_add, add=True)
            return 0

        lax.fori_loop(0, SUBTOK, sub_body, 0)

        def drain_body(j, _):
            pltpu.make_async_copy(
                token_table.at[pl.ds(0, UNIT)], acc_v, sem_add).wait()
            return 0

        lax.fori_loop(0, SUBTOK, drain_body, 0)
        pltpu.sync_copy(acc_v, out_hbm.at[pl.ds(base, UNIT)])
        return 0

    lax.fori_loop(0, n_units, unit_body, 0)


def kernel(tokens, nodes, token_table, node_table):
    tokens = tokens.astype(jnp.int32)
    nodes = nodes.astype(jnp.int32)
    tokens_p = jnp.zeros((N_PAD, SUBTOK), jnp.int32).at[:N_NODES].set(tokens)
    nodes_p = jnp.zeros((N_PAD,), jnp.int32).at[:N_NODES].set(nodes)
    tokens_flat = (tokens_p.reshape(N_UNITS, UNIT, SUBTOK)
                   .transpose(0, 2, 1)
                   .reshape(N_PAD * SUBTOK))
    out = _node_embedding_sc(tokens_flat, nodes_p, token_table, node_table)
    return out[:N_NODES]


# speedup vs baseline: 1.3712x; 1.3712x over previous
import functools

import jax
import jax.numpy as jnp
from jax import lax
from jax.experimental import pallas as pl
from jax.experimental.pallas import tpu as pltpu
from jax.experimental.pallas import tpu_sc as plsc

N_NODES = 50000
SUBTOK = 20
EMB = 128

NC = 2
NS = 16
NW = NC * NS
UNIT = 392
U_TILE = 4
N_UNITS = NW * U_TILE
N_PAD = N_UNITS * UNIT
TAIL = N_NODES - (N_UNITS - 1) * UNIT
IDX_UNIT = UNIT * SUBTOK

_mesh = plsc.VectorSubcoreMesh(core_axis_name="c", subcore_axis_name="s")


@functools.partial(
    pl.kernel,
    out_type=jax.ShapeDtypeStruct((N_NODES, EMB), jnp.float32),
    mesh=_mesh,
    scratch_types=[
        pltpu.VMEM((IDX_UNIT,), jnp.int32),
        pltpu.VMEM((IDX_UNIT,), jnp.int32),
        pltpu.VMEM((UNIT,), jnp.int32),
        pltpu.VMEM((UNIT,), jnp.int32),
        pltpu.VMEM((UNIT, EMB), jnp.float32),
        pltpu.VMEM((UNIT, EMB), jnp.float32),
        pltpu.SemaphoreType.DMA,
        pltpu.SemaphoreType.DMA,
        pltpu.SemaphoreType.DMA,
        pltpu.SemaphoreType.DMA,
    ],
)
def _node_embedding_sc(tokens_hbm, nodes_hbm, token_table, node_table,
                       out_hbm, tok_idx0_v, tok_idx1_v, node_idx0_v,
                       node_idx1_v, acc0_v, acc1_v,
                       sem_add0, sem_add1, sem_init0, sem_init1):
    wid = lax.axis_index("s") * NC + lax.axis_index("c")
    tok_idxs = (tok_idx0_v, tok_idx1_v)
    node_idxs = (node_idx0_v, node_idx1_v)
    accs = (acc0_v, acc1_v)
    sem_adds = (sem_add0, sem_add1)
    sem_inits = (sem_init0, sem_init1)

    def stage_and_init(b, gu):
        pltpu.sync_copy(tokens_hbm.at[pl.ds(gu * IDX_UNIT, IDX_UNIT)],
                        tok_idxs[b])
        pltpu.sync_copy(nodes_hbm.at[pl.ds(gu * UNIT, UNIT)],
                        node_idxs[b])
        pltpu.async_copy(node_table.at[node_idxs[b]], accs[b],
                         sem_inits[b])

    def wait_init(b):
        pltpu.make_async_copy(node_table.at[pl.ds(0, UNIT)], accs[b],
                              sem_inits[b]).wait()

    def fire_adds(b):
        def body(j, _):
            pltpu.async_copy(
                token_table.at[tok_idxs[b].at[pl.ds(j * UNIT, UNIT)]],
                accs[b], sem_adds[b], add=True)
            return 0
        lax.fori_loop(0, SUBTOK, body, 0)

    def drain_adds(b):
        def body(j, _):
            pltpu.make_async_copy(token_table.at[pl.ds(0, UNIT)], accs[b],
                                  sem_adds[b]).wait()
            return 0
        lax.fori_loop(0, SUBTOK, body, 0)

    def writeback(b, gu):
        @pl.when(gu != N_UNITS - 1)
        def _():
            pltpu.sync_copy(accs[b], out_hbm.at[pl.ds(gu * UNIT, UNIT)])

        @pl.when(gu == N_UNITS - 1)
        def _():
            pltpu.sync_copy(accs[b].at[pl.ds(0, TAIL)],
                            out_hbm.at[pl.ds(gu * UNIT, TAIL)])

    stage_and_init(0, wid * U_TILE)

    def pair_body(g, _):
        u0 = wid * U_TILE + 2 * g
        wait_init(0)
        fire_adds(0)
        stage_and_init(1, u0 + 1)
        wait_init(1)
        fire_adds(1)
        drain_adds(0)
        writeback(0, u0)

        @pl.when(g + 1 < U_TILE // 2)
        def _():
            stage_and_init(0, u0 + 2)

        drain_adds(1)
        writeback(1, u0 + 1)
        return 0

    lax.fori_loop(0, U_TILE // 2, pair_body, 0)


def kernel(tokens, nodes, token_table, node_table):
    tokens = tokens.astype(jnp.int32)
    nodes = nodes.astype(jnp.int32)
    tokens_p = jnp.zeros((N_PAD, SUBTOK), jnp.int32).at[:N_NODES].set(tokens)
    nodes_p = jnp.zeros((N_PAD,), jnp.int32).at[:N_NODES].set(nodes)
    tokens_flat = (tokens_p.reshape(N_UNITS, UNIT, SUBTOK)
                   .transpose(0, 2, 1)
                   .reshape(N_PAD * SUBTOK))
    return _node_embedding_sc(tokens_flat, nodes_p, token_table, node_table)


# speedup vs baseline: 1.5156x; 1.1053x over previous
import functools

import jax
import jax.numpy as jnp
from jax import lax
from jax.experimental import pallas as pl
from jax.experimental.pallas import tpu as pltpu
from jax.experimental.pallas import tpu_sc as plsc

N_NODES = 50000
SUBTOK = 20
EMB = 128

NC = 2
NS = 16
NW = NC * NS
UNIT = 224
U_CORE0 = 12
U_CORE1 = 2
N_UNITS = NS * (U_CORE0 + U_CORE1)
N_PAD = N_UNITS * UNIT
TAIL = N_NODES - (N_UNITS - 1) * UNIT
IDX_UNIT = UNIT * SUBTOK

_mesh = plsc.VectorSubcoreMesh(core_axis_name="c", subcore_axis_name="s")


@functools.partial(
    pl.kernel,
    out_type=jax.ShapeDtypeStruct((N_NODES, EMB), jnp.float32),
    mesh=_mesh,
    scratch_types=[
        pltpu.VMEM((IDX_UNIT,), jnp.int32),
        pltpu.VMEM((IDX_UNIT,), jnp.int32),
        pltpu.VMEM((UNIT,), jnp.int32),
        pltpu.VMEM((UNIT,), jnp.int32),
        pltpu.VMEM((UNIT, EMB), jnp.float32),
        pltpu.VMEM((UNIT, EMB), jnp.float32),
        pltpu.SemaphoreType.DMA,
        pltpu.SemaphoreType.DMA,
        pltpu.SemaphoreType.DMA,
        pltpu.SemaphoreType.DMA,
    ],
)
def _node_embedding_sc(tokens_hbm, nodes_hbm, token_table, node_table,
                       out_hbm, tok_idx0_v, tok_idx1_v, node_idx0_v,
                       node_idx1_v, acc0_v, acc1_v,
                       sem_add0, sem_add1, sem_init0, sem_init1):
    cid = lax.axis_index("c")
    sid = lax.axis_index("s")
    n_pairs = lax.select(cid == 0, U_CORE0 // 2, U_CORE1 // 2)
    unit0 = lax.select(cid == 0, sid * U_CORE0,
                       NS * U_CORE0 + sid * U_CORE1)
    tok_idxs = (tok_idx0_v, tok_idx1_v)
    node_idxs = (node_idx0_v, node_idx1_v)
    accs = (acc0_v, acc1_v)
    sem_adds = (sem_add0, sem_add1)
    sem_inits = (sem_init0, sem_init1)

    def stage_and_init(b, gu):
        pltpu.sync_copy(tokens_hbm.at[pl.ds(gu * IDX_UNIT, IDX_UNIT)],
                        tok_idxs[b])
        pltpu.sync_copy(nodes_hbm.at[pl.ds(gu * UNIT, UNIT)],
                        node_idxs[b])
        pltpu.async_copy(node_table.at[node_idxs[b]], accs[b],
                         sem_inits[b])

    def wait_init(b):
        pltpu.make_async_copy(node_table.at[pl.ds(0, UNIT)], accs[b],
                              sem_inits[b]).wait()

    def fire_adds(b):
        def body(j, _):
            pltpu.async_copy(
                token_table.at[tok_idxs[b].at[pl.ds(j * UNIT, UNIT)]],
                accs[b], sem_adds[b], add=True)
            return 0
        lax.fori_loop(0, SUBTOK, body, 0)

    def drain_adds(b):
        def body(j, _):
            pltpu.make_async_copy(token_table.at[pl.ds(0, UNIT)], accs[b],
                                  sem_adds[b]).wait()
            return 0
        lax.fori_loop(0, SUBTOK, body, 0)

    def writeback(b, gu):
        @pl.when(gu != N_UNITS - 1)
        def _():
            pltpu.sync_copy(accs[b], out_hbm.at[pl.ds(gu * UNIT, UNIT)])

        @pl.when(gu == N_UNITS - 1)
        def _():
            pltpu.sync_copy(accs[b].at[pl.ds(0, TAIL)],
                            out_hbm.at[pl.ds(gu * UNIT, TAIL)])

    stage_and_init(0, unit0)

    def pair_body(g, _):
        u0 = unit0 + 2 * g
        wait_init(0)
        fire_adds(0)
        stage_and_init(1, u0 + 1)
        wait_init(1)
        fire_adds(1)
        drain_adds(0)
        writeback(0, u0)

        @pl.when(g + 1 < n_pairs)
        def _():
            stage_and_init(0, u0 + 2)

        drain_adds(1)
        writeback(1, u0 + 1)
        return 0

    lax.fori_loop(0, n_pairs, pair_body, 0)


def kernel(tokens, nodes, token_table, node_table):
    tokens = tokens.astype(jnp.int32)
    nodes = nodes.astype(jnp.int32)
    tokens_p = jnp.zeros((N_PAD, SUBTOK), jnp.int32).at[:N_NODES].set(tokens)
    nodes_p = jnp.zeros((N_PAD,), jnp.int32).at[:N_NODES].set(nodes)
    tokens_flat = (tokens_p.reshape(N_UNITS, UNIT, SUBTOK)
                   .transpose(0, 2, 1)
                   .reshape(N_PAD * SUBTOK))
    return _node_embedding_sc(tokens_flat, nodes_p, token_table, node_table)
